# baseline (device time: 31413 ns/iter reference)
import functools

import jax
import jax.numpy as jnp
from jax import lax
from jax.experimental import pallas as pl
from jax.experimental.pallas import tpu as pltpu

N_DEV = 4
N_LAYERS = 3
N_PEERS = 3
N_SEND = 2 * N_LAYERS
N_SLOTS = N_PEERS * N_SEND


def kernel(x, Win0, Wout0, Win1, Wout1, Win2, Wout2):
    b, d_in = x.shape
    _, h_dim = Win0.shape
    h2 = h_dim // 2

    def body(x_ref, win0_ref, wout0_ref, win1_ref, wout1_ref, win2_ref,
             wout2_ref, out_ref, send_buf, recv_buf, send_sems, recv_sems,
             winv, woutv, win_dma_sems, wout_dma_sems):
        my = lax.axis_index("i")
        peers = [my ^ 1, my ^ 3, my ^ 2]

        wins_hbm = [win0_ref, win1_ref, win2_ref]
        wouts_hbm = [wout0_ref, wout1_ref, wout2_ref]

        def fetch(l):
            win_cp = pltpu.make_async_copy(
                wins_hbm[l], winv.at[l], win_dma_sems.at[l])
            wout_cp = pltpu.make_async_copy(
                wouts_hbm[l], woutv.at[l], wout_dma_sems.at[l])
            win_cp.start()
            wout_cp.start()
            return win_cp, wout_cp

        w_cp = [None] * N_LAYERS
        w_cp[0] = fetch(0)

        barrier_sem = pltpu.get_barrier_semaphore()
        for nbr in peers:
            pl.semaphore_signal(barrier_sem, inc=1, device_id=(nbr,),
                                device_id_type=pl.DeviceIdType.MESH)
        pl.semaphore_wait(barrier_sem, N_PEERS)

        def exchange(send_idx, value):
            send_buf[send_idx] = value
            rdmas = []
            for k in (2, 1, 3):
                slot = send_idx * N_PEERS + (k - 1)
                r = pltpu.make_async_remote_copy(
                    src_ref=send_buf.at[send_idx],
                    dst_ref=recv_buf.at[slot],
                    send_sem=send_sems.at[slot],
                    recv_sem=recv_sems.at[slot],
                    device_id=(my ^ k,),
                    device_id_type=pl.DeviceIdType.MESH,
                )
                r.start()
                rdmas.append(r)
            return rdmas

        def gather(send_idx, partial, rdmas):
            total = partial
            for j, k in enumerate((2, 1, 3)):
                slot = send_idx * N_PEERS + (k - 1)
                rdmas[j].wait_recv()
                total = total + recv_buf[slot].astype(jnp.float32)
            return total

        xs = x_ref[...].astype(jnp.bfloat16)
        for l in range(N_LAYERS):
            iA, iB = 2 * l, 2 * l + 1

            w_cp[l][0].wait()
            win = winv.at[l]
            pA = jnp.dot(xs, win[:, :h2].astype(jnp.bfloat16),
                         preferred_element_type=jnp.float32)
            rA = exchange(iA, pA.astype(jnp.bfloat16))
            if l + 1 < N_LAYERS:
                w_cp[l + 1] = fetch(l + 1)
            pB = jnp.dot(xs, win[:, h2:].astype(jnp.bfloat16),
                         preferred_element_type=jnp.float32)
            rB = exchange(iB, pB.astype(jnp.bfloat16))

            w_cp[l][1].wait()
            wout = woutv.at[l]

            hA = jnp.maximum(gather(iA, pA, rA), 0.0)
            xn = jnp.dot(hA.astype(jnp.bfloat16),
                         wout[:h2, :].astype(jnp.bfloat16),
                         preferred_element_type=jnp.float32)

            hB = jnp.maximum(gather(iB, pB, rB), 0.0)
            xs = (xn + jnp.dot(hB.astype(jnp.bfloat16),
                               wout[h2:, :].astype(jnp.bfloat16),
                               preferred_element_type=jnp.float32)
                  ).astype(jnp.bfloat16)

            for r in rA + rB:
                r.wait_send()
        out_ref[...] = xs.astype(jnp.float32)

        @functools.partial(pl.run_scoped, sem=pltpu.SemaphoreType.REGULAR)
        def _(sem):
            for nbr in peers:
                pl.semaphore_signal(sem, inc=1, device_id=(nbr,),
                                    device_id_type=pl.DeviceIdType.MESH)
            pl.semaphore_wait(sem, N_PEERS)

    return pl.pallas_call(
        body,
        out_shape=jax.ShapeDtypeStruct((b, d_in), jnp.float32),
        in_specs=[pl.BlockSpec(memory_space=pltpu.VMEM)]
        + [pl.BlockSpec(memory_space=pltpu.MemorySpace.HBM)] * 6,
        out_specs=pl.BlockSpec(memory_space=pltpu.VMEM),
        scratch_shapes=[
            pltpu.VMEM((N_SEND, b, h2), jnp.bfloat16),
            pltpu.VMEM((N_SLOTS, b, h2), jnp.bfloat16),
            pltpu.SemaphoreType.DMA((N_SLOTS,)),
            pltpu.SemaphoreType.DMA((N_SLOTS,)),
            pltpu.VMEM((N_LAYERS, d_in, h_dim), jnp.float32),
            pltpu.VMEM((N_LAYERS, h_dim, d_in), jnp.float32),
            pltpu.SemaphoreType.DMA((N_LAYERS,)),
            pltpu.SemaphoreType.DMA((N_LAYERS,)),
        ],
        compiler_params=pltpu.CompilerParams(collective_id=0),
    )(x, Win0, Wout0, Win1, Wout1, Win2, Wout2)


# device time: 31096 ns/iter; 1.0102x vs baseline; 1.0102x over previous
import functools

import jax
import jax.numpy as jnp
from jax import lax
from jax.experimental import pallas as pl
from jax.experimental.pallas import tpu as pltpu

N_DEV = 4
N_LAYERS = 3
N_PEERS = 3
N_SEND = 2 * N_LAYERS
N_SLOTS = N_PEERS * N_SEND


def kernel(x, Win0, Wout0, Win1, Wout1, Win2, Wout2):
    b, d_in = x.shape
    _, h_dim = Win0.shape
    h2 = h_dim // 2

    def body(x_ref, win0_ref, wout0_ref, win1_ref, wout1_ref, win2_ref,
             wout2_ref, out_ref, send_buf, recv_buf, send_sems, recv_sems):
        my = lax.axis_index("i")
        peers = [my ^ 1, my ^ 3, my ^ 2]

        wins = [win0_ref, win1_ref, win2_ref]
        wouts = [wout0_ref, wout1_ref, wout2_ref]

        barrier_sem = pltpu.get_barrier_semaphore()
        for nbr in peers:
            pl.semaphore_signal(barrier_sem, inc=1, device_id=(nbr,),
                                device_id_type=pl.DeviceIdType.MESH)
        pl.semaphore_wait(barrier_sem, N_PEERS)

        def exchange(send_idx, value):
            send_buf[send_idx] = value
            rdmas = []
            for k in (2, 1, 3):
                slot = send_idx * N_PEERS + (k - 1)
                r = pltpu.make_async_remote_copy(
                    src_ref=send_buf.at[send_idx],
                    dst_ref=recv_buf.at[slot],
                    send_sem=send_sems.at[slot],
                    recv_sem=recv_sems.at[slot],
                    device_id=(my ^ k,),
                    device_id_type=pl.DeviceIdType.MESH,
                )
                r.start()
                rdmas.append(r)
            return rdmas

        def gather(send_idx, partial, rdmas):
            total = partial
            for j, k in enumerate((2, 1, 3)):
                slot = send_idx * N_PEERS + (k - 1)
                rdmas[j].wait_recv()
                total = total + recv_buf[slot].astype(jnp.float32)
            return total

        xs = x_ref[...].astype(jnp.bfloat16)
        for l in range(N_LAYERS):
            iA, iB = 2 * l, 2 * l + 1

            win = wins[l]
            pA = jnp.dot(xs, win[:, :h2].astype(jnp.bfloat16),
                         preferred_element_type=jnp.float32)
            rA = exchange(iA, pA.astype(jnp.bfloat16))
            pB = jnp.dot(xs, win[:, h2:].astype(jnp.bfloat16),
                         preferred_element_type=jnp.float32)
            rB = exchange(iB, pB.astype(jnp.bfloat16))

            wout = wouts[l]

            hA = jnp.maximum(gather(iA, pA, rA), 0.0)
            xn = jnp.dot(hA.astype(jnp.bfloat16),
                         wout[:h2, :].astype(jnp.bfloat16),
                         preferred_element_type=jnp.float32)

            hB = jnp.maximum(gather(iB, pB, rB), 0.0)
            xs = (xn + jnp.dot(hB.astype(jnp.bfloat16),
                               wout[h2:, :].astype(jnp.bfloat16),
                               preferred_element_type=jnp.float32)
                  ).astype(jnp.bfloat16)

            for r in rA + rB:
                r.wait_send()
        out_ref[...] = xs.astype(jnp.float32)

        @functools.partial(pl.run_scoped, sem=pltpu.SemaphoreType.REGULAR)
        def _(sem):
            for nbr in peers:
                pl.semaphore_signal(sem, inc=1, device_id=(nbr,),
                                    device_id_type=pl.DeviceIdType.MESH)
            pl.semaphore_wait(sem, N_PEERS)

    return pl.pallas_call(
        body,
        out_shape=jax.ShapeDtypeStruct((b, d_in), jnp.float32),
        in_specs=[pl.BlockSpec(memory_space=pltpu.VMEM)]
        + [pl.BlockSpec(memory_space=pltpu.VMEM)] * 6,
        out_specs=pl.BlockSpec(memory_space=pltpu.VMEM),
        scratch_shapes=[
            pltpu.VMEM((N_SEND, b, h2), jnp.bfloat16),
            pltpu.VMEM((N_SLOTS, b, h2), jnp.bfloat16),
            pltpu.SemaphoreType.DMA((N_SLOTS,)),
            pltpu.SemaphoreType.DMA((N_SLOTS,)),
        ],
        compiler_params=pltpu.CompilerParams(collective_id=0),
    )(x, Win0, Wout0, Win1, Wout1, Win2, Wout2)
